# baseline (speedup 1.0000x reference)
import functools
import math

import jax
import jax.numpy as jnp
from jax import lax
from jax.experimental import pallas as pl
from jax.experimental.pallas import tpu as pltpu
from jax.experimental.pallas import tpu_sc as plsc

NF = 100000
K = 200
H = 64
B = 1024
NW = 32
RPW = B // NW
NVR = NF // 16
HBITS = 12
HSIZE = 1 << HBITS
HSHIFT = 31 - HBITS
CAP = 512
CAPV = CAP // 16

BBLK = 32


def _sc_topk_build():
    mesh = plsc.VectorSubcoreMesh(core_axis_name="c", subcore_axis_name="s")

    @functools.partial(
        pl.kernel,
        mesh=mesh,
        compiler_params=pltpu.CompilerParams(needs_layout_passes=False),
        out_type=[
            jax.ShapeDtypeStruct((B, K), jnp.float32),
            jax.ShapeDtypeStruct((B, K, 128), jnp.float32),
        ],
        scratch_types=[
            pltpu.VMEM((NF,), jnp.float32),
            pltpu.VMEM((HSIZE,), jnp.int32),
            pltpu.VMEM((CAP,), jnp.int32),
            pltpu.VMEM((CAP,), jnp.int32),
            pltpu.VMEM((CAP,), jnp.int32),
            pltpu.VMEM((CAP,), jnp.int32),
            pltpu.VMEM((K,), jnp.float32),
            pltpu.VMEM((104,), jnp.int32),
            pltpu.VMEM((96,), jnp.int32),
            pltpu.VMEM((104, 128), jnp.float32),
            pltpu.SemaphoreType.DMA,
            pltpu.SemaphoreType.DMA,
        ],
    )
    def topk(x_hbm, emb_hbm, val_hbm, tok_hbm,
             xbuf, hist, ck, cs, ci, rk, ov, oia, oib, rows, sem, sem2):
        cid = lax.axis_index("c")
        sid = lax.axis_index("s")
        wid = sid * 2 + cid
        iota = lax.iota(jnp.int32, 16)
        zeros16 = jnp.zeros((16,), jnp.int32)
        ones16 = jnp.ones((16,), jnp.int32)
        def do_row(j, _):
            row = wid * RPW + j

            @plsc.parallel_loop(0, HSIZE // 16, unroll=8)
            def _(t):
                hist[pl.ds(t * 16, 16)] = zeros16

            pltpu.make_async_copy(x_hbm.at[row], xbuf, sem2).wait()

            @plsc.parallel_loop(0, NVR, unroll=16)
            def _(t):
                v = xbuf[pl.ds(t * 16, 16)]
                kb = (lax.bitcast_convert_type(v, jnp.int32)
                      & jnp.int32(0x7FFFFFFF))
                bkt = lax.shift_right_logical(kb, HSHIFT)
                plsc.addupdate_scatter(hist, [bkt], ones16)

            def gcond(c):
                g, acc, accp = c
                return (acc < K) & (g >= 0)

            def gbody(c):
                g, acc, accp = c
                sv = hist[pl.ds(g * 128, 16)]
                for u in range(1, 8):
                    sv = sv + hist[pl.ds(g * 128 + u * 16, 16)]
                return g - 1, acc + jnp.sum(sv), acc

            gf, _, accp = lax.while_loop(
                gcond, gbody,
                (jnp.int32(HSIZE // 128 - 1), jnp.int32(0), jnp.int32(0)))
            gc = gf + 1

            def tcond(c):
                jj, acc, T, ngt = c
                return (acc < K) & (jj >= gc * 8)

            def tbody(c):
                jj, acc, T, ngt = c
                h = hist[pl.ds(jj * 16, 16)]
                r = lax.rev(h, (0,))
                cum = plsc.cumsum(r)
                tot = jnp.sum(h)
                crossing = (acc + tot) >= K
                mk = (acc + cum) >= K
                p = plsc.all_reduce_ffs(mk)[0]
                csel = jnp.sum(jnp.where(iota == p, cum, 0))
                rsel = jnp.sum(jnp.where(iota == p, r, 0))
                t_new = jj * 16 + (15 - p)
                ngt_new = acc + csel - rsel
                T = jnp.where(crossing, t_new, T)
                ngt = jnp.where(crossing, ngt_new, ngt)
                return jj - 1, acc + tot, T, ngt

            _, _, T, ngt = lax.while_loop(
                tcond, tbody,
                (gc * 8 + 7, accp, jnp.int32(0), jnp.int32(0)))

            @plsc.parallel_loop(0, NVR, unroll=8, carry=jnp.int32(0))
            def off(t, off):
                v = xbuf[pl.ds(t * 16, 16)]
                sk = lax.bitcast_convert_type(v, jnp.int32)
                kb = sk & jnp.int32(0x7FFFFFFF)
                msk = lax.shift_right_logical(kb, HSHIFT) >= T
                n = plsc.all_reduce_population_count(msk)[0]
                o = jnp.minimum(off, CAP - 16)
                plsc.store_compressed(ck.at[pl.ds(o, 16)], sk, mask=msk)
                plsc.store_compressed(ci.at[pl.ds(o, 16)], t * 16 + iota,
                                      mask=msk)
                return off + n

            m = jnp.minimum(off, jnp.int32(CAP))

            @pl.when(j < RPW - 1)
            def _():
                pltpu.async_copy(x_hbm.at[row + 1], xbuf, sem2)

            nv = (m + 15) // 16

            @plsc.parallel_loop(0, CAP // 16, unroll=8)
            def _(t):
                cs[pl.ds(t * 16, 16)] = (ck[pl.ds(t * 16, 16)]
                                         & jnp.int32(0x7FFFFFFF))
            cs[pl.ds(jnp.minimum(m, CAP - 16), 16)] = zeros16 - 1

            def outer(jv, _):
                A = cs[pl.ds(jv * 16, 16)]
                rank_vec = zeros16
                for l in range(16):
                    kk = A[l]
                    pos = jv * 16 + l

                    def inner(jb, cnt):
                        Bv = cs[pl.ds(jb * 16, 16)]
                        posB = jb * 16 + iota
                        gt = Bv > kk
                        eq = (Bv == kk) & (posB < pos)
                        return cnt + gt.astype(jnp.int32) + eq.astype(jnp.int32)

                    cnt = lax.fori_loop(0, nv, inner, zeros16)
                    r = jnp.sum(cnt)
                    rank_vec = jnp.where(iota == l, r, rank_vec)
                rk[pl.ds(jv * 16, 16)] = rank_vec
                return 0
            lax.fori_loop(0, nv, outer, 0)

            def scat(jv, _):
                rv = rk[pl.ds(jv * 16, 16)]
                keys = cs[pl.ds(jv * 16, 16)]
                svals = lax.bitcast_convert_type(ck[pl.ds(jv * 16, 16)],
                                                 jnp.float32)
                idxs = ci[pl.ds(jv * 16, 16)]
                posv = jv * 16 + iota
                mm = (rv < K) & (posv < m)
                fid = jnp.where(keys == 0, 0, idxs + 1)
                plsc.store_scatter(ov, [rv], svals, mask=mm)
                ma = mm & (rv < 104)
                mb = mm & (rv >= 104)
                plsc.store_scatter(oia, [rv], fid, mask=ma)
                plsc.store_scatter(oib, [rv - 104], fid, mask=mb)
                return 0
            lax.fori_loop(0, nv, scat, 0)

            pltpu.sync_copy(ov, val_hbm.at[row])
            pltpu.async_copy(emb_hbm.at[oia], rows, sem).wait()
            pltpu.sync_copy(rows, tok_hbm.at[row, pl.ds(0, 104)])
            pltpu.async_copy(emb_hbm.at[oib], rows.at[pl.ds(0, 96)], sem).wait()
            pltpu.sync_copy(rows.at[pl.ds(0, 96)], tok_hbm.at[row, pl.ds(104, 96)])
            return 0

        pltpu.async_copy(x_hbm.at[wid * RPW], xbuf, sem2)
        lax.fori_loop(0, RPW, do_row, 0)

    return topk


_sc_topk = _sc_topk_build()


def _dense_body(vals_ref, tok_ref, w1_ref, b1_ref, w2_ref, b2_ref, g_ref, bt_ref,
                out_ref, am_ref):
    v = vals_ref[...]
    tok = tok_ref[...][:, :, :H]
    w1 = w1_ref[...]
    b1 = b1_ref[...]
    w2 = w2_ref[...]
    b2 = b2_ref[...]
    g = g_ref[...]
    bt = bt_ref[...]

    h = v[:, :, None] * w1[0][None, None, :] + b1[0][None, None, :]
    h = 0.5 * h * (1.0 + jax.lax.erf(h * (1.0 / math.sqrt(2.0))))
    ve = jax.lax.dot_general(h, w2, (((2,), (1,)), ((), ())),
                             preferred_element_type=jnp.float32)
    z = tok + ve + b2[0][None, None, :]
    mean = jnp.mean(z, axis=-1, keepdims=True)
    zc = z - mean
    var = jnp.mean(zc * zc, axis=-1, keepdims=True)
    normed = zc * jax.lax.rsqrt(var + 1e-5) * g[0][None, None, :] + bt[0][None, None, :]

    active = (v != 0.0)
    am = active.astype(jnp.int32)
    empty = jnp.sum(am, axis=1) == 0
    pos0 = jax.lax.broadcasted_iota(jnp.int32, (BBLK, K, 1), 1) == 0
    kill = empty[:, None, None] & pos0
    out_ref[...] = jnp.where(kill, 0.0, normed)
    am_ref[...] = jnp.where(empty[:, None] & (pos0[:, :, 0]), 1, am)


def _dense_stage(vals, tok, W1c, b1, W2, b2, gamma, beta):
    grid = (B // BBLK,)
    out, am = pl.pallas_call(
        _dense_body,
        grid=grid,
        in_specs=[
            pl.BlockSpec((BBLK, K), lambda i: (i, 0)),
            pl.BlockSpec((BBLK, K, 128), lambda i: (i, 0, 0)),
            pl.BlockSpec((1, H), lambda i: (0, 0)),
            pl.BlockSpec((1, H), lambda i: (0, 0)),
            pl.BlockSpec((H, H), lambda i: (0, 0)),
            pl.BlockSpec((1, H), lambda i: (0, 0)),
            pl.BlockSpec((1, H), lambda i: (0, 0)),
            pl.BlockSpec((1, H), lambda i: (0, 0)),
        ],
        out_specs=[
            pl.BlockSpec((BBLK, K, H), lambda i: (i, 0, 0)),
            pl.BlockSpec((BBLK, K), lambda i: (i, 0)),
        ],
        out_shape=[
            jax.ShapeDtypeStruct((B, K, H), jnp.float32),
            jax.ShapeDtypeStruct((B, K), jnp.int32),
        ],
    )(vals, tok, W1c, b1, W2, b2, gamma, beta)
    return out, am


def kernel(x, emb, W1, b1, W2, b2, gamma, beta):
    embp = jnp.pad(emb, ((0, 0), (0, 128 - H)))
    vals, tok = _sc_topk(x, embp)
    out, am = _dense_stage(
        vals, tok,
        W1.reshape(1, H), b1.reshape(1, H), W2, b2.reshape(1, H),
        gamma.reshape(1, H), beta.reshape(1, H))
    return out, am.astype(jnp.int64)


# speedup vs baseline: 1.2328x; 1.2328x over previous
import functools
import math

import jax
import jax.numpy as jnp
from jax import lax
from jax.experimental import pallas as pl
from jax.experimental.pallas import tpu as pltpu
from jax.experimental.pallas import tpu_sc as plsc

NF = 100000
K = 200
H = 64
B = 1024
NW = 32
RPW = B // NW
NVR = NF // 16
HBITS = 12
HSIZE = 1 << HBITS
HSHIFT = 31 - HBITS
CAP = 512
CAPV = CAP // 16

BBLK = 32


def _sc_topk_build():
    mesh = plsc.VectorSubcoreMesh(core_axis_name="c", subcore_axis_name="s")

    @functools.partial(
        pl.kernel,
        mesh=mesh,
        compiler_params=pltpu.CompilerParams(needs_layout_passes=False),
        out_type=[
            jax.ShapeDtypeStruct((B, K), jnp.float32),
            jax.ShapeDtypeStruct((B, K, 128), jnp.float32),
        ],
        scratch_types=[
            pltpu.VMEM((NF,), jnp.float32),
            pltpu.VMEM((HSIZE,), jnp.int32),
            pltpu.VMEM((CAP,), jnp.int32),
            pltpu.VMEM((CAP,), jnp.int32),
            pltpu.VMEM((CAP,), jnp.int32),
            pltpu.VMEM((CAP,), jnp.int32),
            pltpu.VMEM((K,), jnp.float32),
            pltpu.VMEM((104,), jnp.int32),
            pltpu.VMEM((96,), jnp.int32),
            pltpu.VMEM((104, 128), jnp.float32),
            pltpu.SemaphoreType.DMA,
            pltpu.SemaphoreType.DMA,
        ],
    )
    def topk(x_hbm, emb_hbm, val_hbm, tok_hbm,
             xbuf, hist, ck, cs, ci, rk, ov, oia, oib, rows, sem, sem2):
        cid = lax.axis_index("c")
        sid = lax.axis_index("s")
        wid = sid * 2 + cid
        iota = lax.iota(jnp.int32, 16)
        zeros16 = jnp.zeros((16,), jnp.int32)
        ones16 = jnp.ones((16,), jnp.int32)
        def do_row(j, _):
            row = wid * RPW + j

            @plsc.parallel_loop(0, HSIZE // 16, unroll=8)
            def _(t):
                hist[pl.ds(t * 16, 16)] = zeros16

            pltpu.make_async_copy(x_hbm.at[row], xbuf, sem2).wait()

            @plsc.parallel_loop(0, NVR, unroll=8)
            def _(t):
                v = xbuf[pl.ds(t * 16, 16)]
                kb = (lax.bitcast_convert_type(v, jnp.int32)
                      & jnp.int32(0x7FFFFFFF))
                bkt = lax.shift_right_logical(kb, HSHIFT)
                plsc.addupdate_scatter(hist, [bkt], ones16)

            def gcond(c):
                g, acc, accp = c
                return (acc < K) & (g >= 0)

            def gbody(c):
                g, acc, accp = c
                sv = hist[pl.ds(g * 128, 16)]
                for u in range(1, 8):
                    sv = sv + hist[pl.ds(g * 128 + u * 16, 16)]
                return g - 1, acc + jnp.sum(sv), acc

            gf, _, accp = lax.while_loop(
                gcond, gbody,
                (jnp.int32(HSIZE // 128 - 1), jnp.int32(0), jnp.int32(0)))
            gc = gf + 1

            def tcond(c):
                jj, acc, T, ngt = c
                return (acc < K) & (jj >= gc * 8)

            def tbody(c):
                jj, acc, T, ngt = c
                h = hist[pl.ds(jj * 16, 16)]
                r = lax.rev(h, (0,))
                cum = plsc.cumsum(r)
                tot = jnp.sum(h)
                crossing = (acc + tot) >= K
                mk = (acc + cum) >= K
                p = plsc.all_reduce_ffs(mk)[0]
                csel = jnp.sum(jnp.where(iota == p, cum, 0))
                rsel = jnp.sum(jnp.where(iota == p, r, 0))
                t_new = jj * 16 + (15 - p)
                ngt_new = acc + csel - rsel
                T = jnp.where(crossing, t_new, T)
                ngt = jnp.where(crossing, ngt_new, ngt)
                return jj - 1, acc + tot, T, ngt

            _, _, T, ngt = lax.while_loop(
                tcond, tbody,
                (gc * 8 + 7, accp, jnp.int32(0), jnp.int32(0)))

            @plsc.parallel_loop(0, NVR, unroll=4, carry=jnp.int32(0))
            def off(t, off):
                v = xbuf[pl.ds(t * 16, 16)]
                sk = lax.bitcast_convert_type(v, jnp.int32)
                kb = sk & jnp.int32(0x7FFFFFFF)
                msk = lax.shift_right_logical(kb, HSHIFT) >= T
                n = plsc.all_reduce_population_count(msk)[0]
                o = jnp.minimum(off, CAP - 16)
                plsc.store_compressed(ck.at[pl.ds(o, 16)], sk, mask=msk)
                plsc.store_compressed(ci.at[pl.ds(o, 16)], t * 16 + iota,
                                      mask=msk)
                return off + n

            m = jnp.minimum(off, jnp.int32(CAP))

            @pl.when(j < RPW - 1)
            def _():
                pltpu.async_copy(x_hbm.at[row + 1], xbuf, sem2)

            nv = (m + 15) // 16

            @plsc.parallel_loop(0, CAP // 16, unroll=8)
            def _(t):
                cs[pl.ds(t * 16, 16)] = (ck[pl.ds(t * 16, 16)]
                                         & jnp.int32(0x7FFFFFFF))
            cs[pl.ds(jnp.minimum(m, CAP - 16), 16)] = zeros16 - 1

            def outer(jv, _):
                A = cs[pl.ds(jv * 16, 16)]
                rank_vec = zeros16
                for l0 in range(0, 16, 4):
                    kks = [A[l0 + d] for d in range(4)]
                    poss = [jv * 16 + l0 + d for d in range(4)]

                    def inner(jb, cnts):
                        Bv = cs[pl.ds(jb * 16, 16)]
                        posB = jb * 16 + iota
                        out = []
                        for d in range(4):
                            gt = Bv > kks[d]
                            eq = (Bv == kks[d]) & (posB < poss[d])
                            out.append(cnts[d] + gt.astype(jnp.int32)
                                       + eq.astype(jnp.int32))
                        return tuple(out)

                    cnts = lax.fori_loop(0, nv, inner,
                                         (zeros16, zeros16, zeros16, zeros16))
                    for d in range(4):
                        rank_vec = jnp.where(iota == l0 + d,
                                             jnp.sum(cnts[d]), rank_vec)
                rk[pl.ds(jv * 16, 16)] = rank_vec
                return 0
            lax.fori_loop(0, nv, outer, 0)

            def scat(jv, _):
                rv = rk[pl.ds(jv * 16, 16)]
                keys = cs[pl.ds(jv * 16, 16)]
                svals = lax.bitcast_convert_type(ck[pl.ds(jv * 16, 16)],
                                                 jnp.float32)
                idxs = ci[pl.ds(jv * 16, 16)]
                posv = jv * 16 + iota
                mm = (rv < K) & (posv < m)
                fid = jnp.where(keys == 0, 0, idxs + 1)
                plsc.store_scatter(ov, [rv], svals, mask=mm)
                ma = mm & (rv < 104)
                mb = mm & (rv >= 104)
                plsc.store_scatter(oia, [rv], fid, mask=ma)
                plsc.store_scatter(oib, [rv - 104], fid, mask=mb)
                return 0
            lax.fori_loop(0, nv, scat, 0)

            pltpu.sync_copy(ov, val_hbm.at[row])
            pltpu.async_copy(emb_hbm.at[oia], rows, sem).wait()
            pltpu.sync_copy(rows, tok_hbm.at[row, pl.ds(0, 104)])
            pltpu.async_copy(emb_hbm.at[oib], rows.at[pl.ds(0, 96)], sem).wait()
            pltpu.sync_copy(rows.at[pl.ds(0, 96)], tok_hbm.at[row, pl.ds(104, 96)])
            return 0

        pltpu.async_copy(x_hbm.at[wid * RPW], xbuf, sem2)
        lax.fori_loop(0, RPW, do_row, 0)

    return topk


_sc_topk = _sc_topk_build()


def _dense_body(vals_ref, tok_ref, w1_ref, b1_ref, w2_ref, b2_ref, g_ref, bt_ref,
                out_ref, am_ref):
    v = vals_ref[...]
    tok = tok_ref[...][:, :, :H]
    w1 = w1_ref[...]
    b1 = b1_ref[...]
    w2 = w2_ref[...]
    b2 = b2_ref[...]
    g = g_ref[...]
    bt = bt_ref[...]

    h = v[:, :, None] * w1[0][None, None, :] + b1[0][None, None, :]
    h = 0.5 * h * (1.0 + jax.lax.erf(h * (1.0 / math.sqrt(2.0))))
    ve = jax.lax.dot_general(h, w2, (((2,), (1,)), ((), ())),
                             preferred_element_type=jnp.float32)
    z = tok + ve + b2[0][None, None, :]
    mean = jnp.mean(z, axis=-1, keepdims=True)
    zc = z - mean
    var = jnp.mean(zc * zc, axis=-1, keepdims=True)
    normed = zc * jax.lax.rsqrt(var + 1e-5) * g[0][None, None, :] + bt[0][None, None, :]

    active = (v != 0.0)
    am = active.astype(jnp.int32)
    empty = jnp.sum(am, axis=1) == 0
    pos0 = jax.lax.broadcasted_iota(jnp.int32, (BBLK, K, 1), 1) == 0
    kill = empty[:, None, None] & pos0
    out_ref[...] = jnp.where(kill, 0.0, normed)
    am_ref[...] = jnp.where(empty[:, None] & (pos0[:, :, 0]), 1, am)


def _dense_stage(vals, tok, W1c, b1, W2, b2, gamma, beta):
    grid = (B // BBLK,)
    out, am = pl.pallas_call(
        _dense_body,
        grid=grid,
        in_specs=[
            pl.BlockSpec((BBLK, K), lambda i: (i, 0)),
            pl.BlockSpec((BBLK, K, 128), lambda i: (i, 0, 0)),
            pl.BlockSpec((1, H), lambda i: (0, 0)),
            pl.BlockSpec((1, H), lambda i: (0, 0)),
            pl.BlockSpec((H, H), lambda i: (0, 0)),
            pl.BlockSpec((1, H), lambda i: (0, 0)),
            pl.BlockSpec((1, H), lambda i: (0, 0)),
            pl.BlockSpec((1, H), lambda i: (0, 0)),
        ],
        out_specs=[
            pl.BlockSpec((BBLK, K, H), lambda i: (i, 0, 0)),
            pl.BlockSpec((BBLK, K), lambda i: (i, 0)),
        ],
        out_shape=[
            jax.ShapeDtypeStruct((B, K, H), jnp.float32),
            jax.ShapeDtypeStruct((B, K), jnp.int32),
        ],
    )(vals, tok, W1c, b1, W2, b2, gamma, beta)
    return out, am


def kernel(x, emb, W1, b1, W2, b2, gamma, beta):
    embp = jnp.pad(emb, ((0, 0), (0, 128 - H)))
    vals, tok = _sc_topk(x, embp)
    out, am = _dense_stage(
        vals, tok,
        W1.reshape(1, H), b1.reshape(1, H), W2, b2.reshape(1, H),
        gamma.reshape(1, H), beta.reshape(1, H))
    return out, am.astype(jnp.int64)
